# Initial kernel scaffold; baseline (speedup 1.0000x reference)
import jax
import jax.numpy as jnp
from jax.experimental import pallas as pl


def kernel(inputs, W, gamma, beta, noise):
    raise NotImplementedError("write your pallas kernel here")


# speedup vs baseline: 1.8817x; 1.8817x over previous
import math

import jax
import jax.numpy as jnp
from jax.experimental import pallas as pl
from jax.experimental.pallas import tpu as pltpu

_NOISE_STD = 1.0
_GSHARD_W = 0.0
_IMP_W = 1.0
_LOAD_W = 1.0


def _router_kernel(x_ref, w_ref, gamma_ref, beta_ref, noise_ref,
                   gates_out_ref, stats_ref,
                   imp_acc, mg_acc, cnt_acc, lsum_acc, lsq_acc):
    i = pl.program_id(0)
    nsteps = pl.num_programs(0)
    g, bs, d = x_ref.shape
    e = w_ref.shape[0]
    rows = g * bs
    noise_std = max(1.0 / e * _NOISE_STD, 1e-6)

    x = x_ref[...]
    mu = jnp.mean(x, axis=-1, keepdims=True)
    var = jnp.mean((x - mu) ** 2, axis=-1, keepdims=True)
    xn = (x - mu) / jnp.sqrt(var + 1e-5) * gamma_ref[...] + beta_ref[...]

    xm = xn.reshape(rows, d)
    logits = jax.lax.dot_general(
        xm, w_ref[...], (((1,), (1,)), ((), ())),
        preferred_element_type=jnp.float32,
        precision=jax.lax.Precision.HIGHEST)
    logits = logits - jnp.max(logits, axis=-1, keepdims=True)

    eg = jnp.exp(logits)
    gates = eg / jnp.sum(eg, axis=-1, keepdims=True)

    ln = logits + noise_std * noise_ref[...].reshape(rows, e)
    en = jnp.exp(ln - jnp.max(ln, axis=-1, keepdims=True))
    gates_noisy = en / jnp.sum(en, axis=-1, keepdims=True)
    gates_out_ref[...] = gates_noisy.reshape(g, bs, e)

    iota = jax.lax.broadcasted_iota(jnp.int32, (rows, e), 1)
    m1 = jnp.max(ln, axis=-1, keepdims=True)
    a1 = jnp.min(jnp.where(ln >= m1, iota, e), axis=-1, keepdims=True)
    thr = jnp.max(jnp.where(iota == a1, -jnp.inf, ln), axis=-1, keepdims=True)
    nrw = jnp.clip((thr - logits) / noise_std, -10.0, 10.0)
    p = 0.5 * (1.0 + jax.lax.erf(nrw * (1.0 / math.sqrt(2.0))))
    pm = jnp.mean(p.reshape(g, bs, e), axis=0)

    imp_part = jnp.sum(gates.reshape(g, bs, e), axis=1)
    mg_part = jnp.sum(gates_noisy, axis=0, keepdims=True)
    cnt_part = jnp.sum((iota == a1).astype(jnp.float32), axis=0,
                       keepdims=True)
    lsum_part = jnp.sum(pm)
    lsq_part = jnp.sum(pm * pm)

    @pl.when(i == 0)
    def _():
        imp_acc[...] = imp_part
        mg_acc[...] = mg_part
        cnt_acc[...] = cnt_part
        lsum_acc[0, 0] = lsum_part
        lsq_acc[0, 0] = lsq_part

    @pl.when(i > 0)
    def _():
        imp_acc[...] += imp_part
        mg_acc[...] += mg_part
        cnt_acc[...] += cnt_part
        lsum_acc[0, 0] += lsum_part
        lsq_acc[0, 0] += lsq_part

    @pl.when(i == nsteps - 1)
    def _():
        n_tok = jnp.float32(g * bs * nsteps)
        imp = imp_acc[...]
        imp_mean = jnp.mean(imp, axis=1, keepdims=True)
        imp_var = jnp.sum((imp - imp_mean) ** 2, axis=1,
                          keepdims=True) / (e - 1)
        imp_loss = jnp.mean(imp_var / (imp_mean * imp_mean))

        mean_t = cnt_acc[...] / n_tok
        mean_g = mg_acc[...] / n_tok
        gshard = jnp.mean(mean_t * mean_g) * (e * e)

        m = jnp.float32(bs * nsteps * e)
        pm_mean = lsum_acc[0, 0] / m
        pm_var = lsq_acc[0, 0] / m - pm_mean * pm_mean
        load = pm_var / (pm_mean * pm_mean)

        stats_ref[0, 0] = _GSHARD_W * gshard + _IMP_W * imp_loss + _LOAD_W * load
        stats_ref[0, 1] = gshard
        stats_ref[0, 2] = imp_loss
        stats_ref[0, 3] = load


def kernel(inputs, W, gamma, beta, noise):
    g, s, d = inputs.shape
    e = W.shape[0]
    bs = 512
    grid = (s // bs,)

    gates_noisy, stats = pl.pallas_call(
        _router_kernel,
        grid=grid,
        in_specs=[
            pl.BlockSpec((g, bs, d), lambda i: (0, i, 0)),
            pl.BlockSpec((e, d), lambda i: (0, 0)),
            pl.BlockSpec((1, d), lambda i: (0, 0)),
            pl.BlockSpec((1, d), lambda i: (0, 0)),
            pl.BlockSpec((g, bs, e), lambda i: (0, i, 0)),
        ],
        out_specs=[
            pl.BlockSpec((g, bs, e), lambda i: (0, i, 0)),
            pl.BlockSpec(memory_space=pltpu.SMEM),
        ],
        out_shape=[
            jax.ShapeDtypeStruct((g, s, e), jnp.float32),
            jax.ShapeDtypeStruct((1, 4), jnp.float32),
        ],
        scratch_shapes=[
            pltpu.VMEM((g, e), jnp.float32),
            pltpu.VMEM((1, e), jnp.float32),
            pltpu.VMEM((1, e), jnp.float32),
            pltpu.SMEM((1, 1), jnp.float32),
            pltpu.SMEM((1, 1), jnp.float32),
        ],
    )(inputs, W, gamma.reshape(1, d), beta.reshape(1, d), noise)

    return (gates_noisy, stats[0, 0], stats[0, 1], stats[0, 2], stats[0, 3])


# speedup vs baseline: 3.1133x; 1.6545x over previous
import math

import jax
import jax.numpy as jnp
from jax.experimental import pallas as pl
from jax.experimental.pallas import tpu as pltpu

_NOISE_STD = 1.0
_GSHARD_W = 0.0
_IMP_W = 1.0
_LOAD_W = 1.0


def _router_kernel(x_ref, w_ref, gamma_ref, beta_ref, noise_ref,
                   gates_out_ref, stats_ref,
                   imp_acc, mg_acc, cnt_acc, lsum_acc, lsq_acc):
    i = pl.program_id(0)
    nsteps = pl.num_programs(0)
    g, bs, d = x_ref.shape
    e = w_ref.shape[0]
    rows = g * bs
    noise_std = max(1.0 / e * _NOISE_STD, 1e-6)

    x = x_ref[...]
    mu = jnp.mean(x, axis=-1, keepdims=True)
    var = jnp.mean((x - mu) ** 2, axis=-1, keepdims=True)
    xn = (x - mu) / jnp.sqrt(var + 1e-5) * gamma_ref[...] + beta_ref[...]

    xm = xn.reshape(rows, d)
    logits = jax.lax.dot_general(
        xm, w_ref[...], (((1,), (1,)), ((), ())),
        preferred_element_type=jnp.float32)
    logits = logits - jnp.max(logits, axis=-1, keepdims=True)

    eg = jnp.exp(logits)
    gates = eg / jnp.sum(eg, axis=-1, keepdims=True)

    ln = logits + noise_std * noise_ref[...].reshape(rows, e)
    en = jnp.exp(ln - jnp.max(ln, axis=-1, keepdims=True))
    gates_noisy = en / jnp.sum(en, axis=-1, keepdims=True)
    gates_out_ref[...] = gates_noisy.reshape(g, bs, e)

    iota = jax.lax.broadcasted_iota(jnp.int32, (rows, e), 1)
    m1 = jnp.max(ln, axis=-1, keepdims=True)
    a1 = jnp.min(jnp.where(ln >= m1, iota, e), axis=-1, keepdims=True)
    thr = jnp.max(jnp.where(iota == a1, -jnp.inf, ln), axis=-1, keepdims=True)
    nrw = jnp.clip((thr - logits) / noise_std, -10.0, 10.0)
    p = 0.5 * (1.0 + jax.lax.erf(nrw * (1.0 / math.sqrt(2.0))))
    pm = jnp.mean(p.reshape(g, bs, e), axis=0)

    imp_part = jnp.sum(gates.reshape(g, bs, e), axis=1)
    mg_part = jnp.sum(gates_noisy, axis=0, keepdims=True)
    cnt_part = jnp.sum((iota == a1).astype(jnp.float32), axis=0,
                       keepdims=True)
    lsum_part = jnp.sum(pm)
    lsq_part = jnp.sum(pm * pm)

    @pl.when(i == 0)
    def _():
        imp_acc[...] = imp_part
        mg_acc[...] = mg_part
        cnt_acc[...] = cnt_part
        lsum_acc[0, 0] = lsum_part
        lsq_acc[0, 0] = lsq_part

    @pl.when(i > 0)
    def _():
        imp_acc[...] += imp_part
        mg_acc[...] += mg_part
        cnt_acc[...] += cnt_part
        lsum_acc[0, 0] += lsum_part
        lsq_acc[0, 0] += lsq_part

    @pl.when(i == nsteps - 1)
    def _():
        n_tok = jnp.float32(g * bs * nsteps)
        imp = imp_acc[...]
        imp_mean = jnp.mean(imp, axis=1, keepdims=True)
        imp_var = jnp.sum((imp - imp_mean) ** 2, axis=1,
                          keepdims=True) / (e - 1)
        imp_loss = jnp.mean(imp_var / (imp_mean * imp_mean))

        mean_t = cnt_acc[...] / n_tok
        mean_g = mg_acc[...] / n_tok
        gshard = jnp.mean(mean_t * mean_g) * (e * e)

        m = jnp.float32(bs * nsteps * e)
        pm_mean = lsum_acc[0, 0] / m
        pm_var = lsq_acc[0, 0] / m - pm_mean * pm_mean
        load = pm_var / (pm_mean * pm_mean)

        stats_ref[0, 0] = _GSHARD_W * gshard + _IMP_W * imp_loss + _LOAD_W * load
        stats_ref[0, 1] = gshard
        stats_ref[0, 2] = imp_loss
        stats_ref[0, 3] = load


def kernel(inputs, W, gamma, beta, noise):
    g, s, d = inputs.shape
    e = W.shape[0]
    bs = 512
    grid = (s // bs,)

    gates_noisy, stats = pl.pallas_call(
        _router_kernel,
        grid=grid,
        in_specs=[
            pl.BlockSpec((g, bs, d), lambda i: (0, i, 0)),
            pl.BlockSpec((e, d), lambda i: (0, 0)),
            pl.BlockSpec((1, d), lambda i: (0, 0)),
            pl.BlockSpec((1, d), lambda i: (0, 0)),
            pl.BlockSpec((g, bs, e), lambda i: (0, i, 0)),
        ],
        out_specs=[
            pl.BlockSpec((g, bs, e), lambda i: (0, i, 0)),
            pl.BlockSpec(memory_space=pltpu.SMEM),
        ],
        out_shape=[
            jax.ShapeDtypeStruct((g, s, e), jnp.float32),
            jax.ShapeDtypeStruct((1, 4), jnp.float32),
        ],
        scratch_shapes=[
            pltpu.VMEM((g, e), jnp.float32),
            pltpu.VMEM((1, e), jnp.float32),
            pltpu.VMEM((1, e), jnp.float32),
            pltpu.SMEM((1, 1), jnp.float32),
            pltpu.SMEM((1, 1), jnp.float32),
        ],
    )(inputs, W, gamma.reshape(1, d), beta.reshape(1, d), noise)

    return (gates_noisy, stats[0, 0], stats[0, 1], stats[0, 2], stats[0, 3])
